# baseline (speedup 1.0000x reference)
import functools

import jax
import jax.numpy as jnp
from jax import lax
from jax.experimental import pallas as pl
from jax.experimental.pallas import tpu as pltpu

LEAK_FACTOR = 0.2
NUM_TOPICS = 20
HIDDEN = 10
OUT = 4
BN_EPS = 1e-5
LANE = 128

PROWS = 32
PCOLS = 128


def _round_up(x, m):
    return (x + m - 1) // m * m


def _hidden_fm(x_ref, p_ref):
    x = x_ref[...]
    w1 = p_ref[0:NUM_TOPICS, 0:HIDDEN]
    b1 = p_ref[0:HIDDEN, 120:121]
    h = lax.dot_general(w1, x, (((0,), (1,)), ((), ())),
                        preferred_element_type=jnp.float32)
    h = h + b1
    return jnp.maximum(h, LEAK_FACTOR * h)


def _stats_kernel(x_ref, p_ref, o_ref, *, batch, tile_rows):
    h = _hidden_fm(x_ref, p_ref)

    def emit(hv):
        o_ref[...] = jnp.zeros_like(o_ref)
        o_ref[0:HIDDEN, 0:1] = jnp.sum(hv, axis=1, keepdims=True)
        o_ref[0:HIDDEN, 1:2] = jnp.sum(hv * hv, axis=1, keepdims=True)

    tail = batch % tile_rows
    if tail == 0:
        emit(h)
    else:
        is_last = pl.program_id(0) == pl.num_programs(0) - 1

        @pl.when(jnp.logical_not(is_last))
        def _():
            emit(h)

        @pl.when(is_last)
        def _():
            lane = lax.broadcasted_iota(jnp.int32, h.shape, 1)
            emit(jnp.where(lane < tail, h, 0.0))


def _apply_kernel(x_ref, p_ref, o_ref):
    h = _hidden_fm(x_ref, p_ref)
    w2pt = p_ref[0:HIDDEN, 100:100 + OUT]
    b2p = p_ref[0:OUT, 110:111]
    logits = lax.dot_general(w2pt, h, (((0,), (0,)), ((), ())),
                             preferred_element_type=jnp.float32)
    logits = logits + b2p
    m = jnp.max(logits, axis=0, keepdims=True)
    e = jnp.exp(logits - m)
    denom = jnp.sum(e, axis=0, keepdims=True)
    s = e / denom
    eye4 = p_ref[0:OUT, 112:112 + OUT]
    o_ref[...] = lax.dot_general(s, eye4, (((0,), (0,)), ((), ())),
                                 preferred_element_type=jnp.float32)


def _pack_base(w1, b1):
    p = jnp.zeros((PROWS, PCOLS), jnp.float32)
    p = p.at[0:NUM_TOPICS, 0:HIDDEN].set(jnp.asarray(w1, jnp.float32))
    p = p.at[0:HIDDEN, 120].set(jnp.asarray(b1, jnp.float32).reshape(-1))
    p = p.at[0:OUT, 112:112 + OUT].set(jnp.eye(OUT, dtype=jnp.float32))
    return p


def kernel(noise, w1, b1, gamma, beta, w2, b2, *, block_rows=8192):
    B = noise.shape[0]
    tb = max(LANE, min(_round_up(block_rows, LANE), _round_up(B, LANE)))
    bp = _round_up(B, tb)
    nbt = bp // tb

    x = jnp.asarray(noise, jnp.float32)
    if bp != B:
        x = jnp.pad(x, ((0, bp - B), (0, 0)))

    p_base = _pack_base(w1, b1)

    compiler_params = pltpu.CompilerParams(
        dimension_semantics=("parallel",),
        vmem_limit_bytes=64 * 1024 * 1024,
    )
    in_specs = [
        pl.BlockSpec((tb, NUM_TOPICS), lambda i: (i, 0)),
        pl.BlockSpec((PROWS, PCOLS), lambda i: (0, 0)),
    ]

    stats = pl.pallas_call(
        functools.partial(_stats_kernel, batch=B, tile_rows=tb),
        out_shape=jax.ShapeDtypeStruct((PROWS, nbt * LANE), jnp.float32),
        grid=(nbt,),
        in_specs=in_specs,
        out_specs=pl.BlockSpec((PROWS, LANE), lambda i: (0, i)),
        cost_estimate=pl.CostEstimate(
            flops=2 * bp * NUM_TOPICS * HIDDEN + 6 * bp * HIDDEN,
            transcendentals=0,
            bytes_accessed=4 * (NUM_TOPICS * bp + PROWS * PCOLS
                                + PROWS * LANE * nbt)),
        compiler_params=compiler_params,
    )(x, p_base)

    stats = stats.reshape(PROWS, nbt, LANE)
    sums = jnp.sum(stats[:HIDDEN, :, 0], axis=1)
    sqs = jnp.sum(stats[:HIDDEN, :, 1], axis=1)
    mean = sums / B
    var = jnp.maximum(sqs / B - mean * mean, 0.0)
    scale = jnp.asarray(gamma, jnp.float32).reshape(-1) * lax.rsqrt(var + BN_EPS)
    shift = jnp.asarray(beta, jnp.float32).reshape(-1) - mean * scale
    w2t = jnp.asarray(w2, jnp.float32).T
    w2p = w2t * scale[None, :]
    b2p = jnp.asarray(b2, jnp.float32).reshape(-1) + w2t @ shift
    p_apply = p_base.at[0:HIDDEN, 100:100 + OUT].set(w2p.T)
    p_apply = p_apply.at[0:OUT, 110].set(b2p)

    out = pl.pallas_call(
        _apply_kernel,
        out_shape=jax.ShapeDtypeStruct((bp, OUT), jnp.float32),
        grid=(nbt,),
        in_specs=in_specs,
        out_specs=pl.BlockSpec((tb, OUT), lambda i: (i, 0)),
        cost_estimate=pl.CostEstimate(
            flops=2 * bp * (NUM_TOPICS * HIDDEN + HIDDEN * OUT + OUT * OUT)
                  + 12 * bp * OUT,
            transcendentals=bp * OUT,
            bytes_accessed=4 * (NUM_TOPICS * bp + PROWS * PCOLS + OUT * bp)),
        compiler_params=compiler_params,
    )(x, p_apply)

    return out[:B]
